# baseline (speedup 1.0000x reference)
import jax
import jax.numpy as jnp
from jax import lax
from jax.experimental import pallas as pl
from jax.experimental.pallas import tpu as pltpu

_N = 4096
_D = 1024
_TOPK = 1024
_SC = 512
_RC = 128
_MC = 512


def _body(x_ref, p_ref, out_ref, ycol_scr, rank_scr, nrm_scr):
    p_row = p_ref[...]
    nrm_scr[...] = jnp.sqrt(jnp.sum(p_row * p_row)).reshape(1, 1)
    nrm = nrm_scr[0, 0]

    p_col = p_row.reshape(_D, 1)

    def score_chunk(c, carry):
        base = pl.multiple_of(c * _SC, _SC)
        xc = x_ref[0, pl.ds(base, _SC), :]
        yc = None
        for q in range(4):
            dq = lax.dot_general(xc[:, q * 256:(q + 1) * 256],
                                 p_col[q * 256:(q + 1) * 256, :],
                                 (((1,), (0,)), ((), ())),
                                 preferred_element_type=jnp.float32)
            yc = dq if yc is None else yc + dq
        ycol_scr[pl.ds(base, _SC), :] = yc / nrm
        return carry

    lax.fori_loop(0, _N // _SC, score_chunk, 0)

    y_col = ycol_scr[...]
    n_i = lax.broadcasted_iota(jnp.int32, (_N, 128), 0)
    l_i = lax.broadcasted_iota(jnp.int32, (_N, 128), 1)
    z = jnp.where((n_i & 127) == l_i, 1.0, 0.0) * y_col
    s_i = lax.broadcasted_iota(jnp.int32, (32, _N), 0)
    n2_i = lax.broadcasted_iota(jnp.int32, (32, _N), 1)
    a32 = jnp.where((n2_i >> 7) == s_i, 1.0, 0.0)
    y32 = lax.dot_general(a32, z, (((1,), (0,)), ((), ())),
                          preferred_element_type=jnp.float32,
                          precision=jax.lax.Precision.HIGHEST)
    y3 = y32.reshape(1, 32, 128)
    col3 = (lax.broadcasted_iota(jnp.int32, (1, 32, 128), 1) * 128
            + lax.broadcasted_iota(jnp.int32, (1, 32, 128), 2))

    def rank_chunk(i, carry):
        base = pl.multiple_of(i * _RC, _RC)
        yr = ycol_scr[pl.ds(base, _RC), :].reshape(_RC, 1, 1)
        rid = lax.broadcasted_iota(jnp.int32, (_RC, 1, 1), 0) + i * _RC
        gt = y3 > yr
        tie = (y3 == yr) & (col3 < rid)
        cnt = jnp.sum(jnp.sum(jnp.where(gt | tie, 1.0, 0.0), axis=2), axis=1)
        rank_scr[pl.ds(base, _RC), :] = cnt.reshape(_RC, 1)
        return carry

    lax.fori_loop(0, _N // _RC, rank_chunk, 0)

    kio = lax.broadcasted_iota(jnp.int32, (1, _TOPK), 1).astype(jnp.float32)
    out_ref[0] = jnp.zeros((_TOPK, _D), jnp.float32)

    def mm_chunk(c, carry):
        base = pl.multiple_of(c * _MC, _MC)
        rk = rank_scr[pl.ds(base, _MC), :]
        gv = jnp.tanh(ycol_scr[pl.ds(base, _MC), :])
        pct = jnp.where(rk == kio, gv, 0.0).astype(jnp.bfloat16)
        xc = x_ref[0, pl.ds(base, _MC), :].astype(jnp.bfloat16)
        out_ref[0] += lax.dot_general(pct, xc, (((0,), (0,)), ((), ())),
                                      preferred_element_type=jnp.float32)
        return carry

    lax.fori_loop(0, _N // _MC, mm_chunk, 0)


def kernel(inputs, p):
    b = inputs.shape[0]
    p2 = p.reshape(1, _D)
    return pl.pallas_call(
        _body,
        grid=(b,),
        in_specs=[
            pl.BlockSpec((1, _N, _D), lambda i: (i, 0, 0)),
            pl.BlockSpec((1, _D), lambda i: (0, 0)),
        ],
        out_specs=pl.BlockSpec((1, _TOPK, _D), lambda i: (i, 0, 0)),
        out_shape=jax.ShapeDtypeStruct((b, _TOPK, _D), jnp.float32),
        scratch_shapes=[
            pltpu.VMEM((_N, 1), jnp.float32),
            pltpu.VMEM((_N, 1), jnp.float32),
            pltpu.VMEM((1, 1), jnp.float32),
        ],
    )(inputs, p2)
